# baseline (speedup 1.0000x reference)
import functools

import jax
import jax.numpy as jnp
import numpy as np
from jax import lax
from jax.experimental import pallas as pl
from jax.experimental.pallas import tpu as pltpu
from jax.experimental.pallas import tpu_sc as plsc

D = 2048
NS = 16
SEQ_IMG = 729
N_TOK = 4097
SEQ_OUT = N_TOK + SEQ_IMG
IMG_BASE = 2049

NW = 32
SUB = 16
NCHUNK = 8
NBUF = 3
IMG_SUB = 23
IMG_HALF = 3

def _sc_body(table_hbm, img_hbm, tok_src_hbm, out_hbm, sidx_v,
             didx_v, buf0, buf1, buf2, ibuf0, si0, si1, si2,
             so0, so1, so2, smi0):
    c = lax.axis_index("c")
    s = lax.axis_index("s")
    w = c * 16 + s

    pltpu.sync_copy(tok_src_hbm.at[w], sidx_v)

    bufs = (buf0, buf1, buf2)
    sin = (si0, si1, si2)
    sout = (so0, so1, so2)

    ib = jnp.minimum(w * IMG_SUB, SEQ_IMG - IMG_SUB)

    gat = [None] * NCHUNK
    sca = [None] * NCHUNK
    for k in range(min(NBUF, NCHUNK)):
        gat[k] = pltpu.async_copy(
            table_hbm.at[sidx_v.at[k]], bufs[k % NBUF], sin[k % NBUF])

    dst0 = jnp.where(w < 16, w * 128, 2778 + (w - 16) * 128)
    lane = lax.iota(jnp.int32, NS) * NS
    for j in range(NCHUNK):
        base = NS * (dst0 + j * SUB) + lane
        for cs in range(NS):
            didx_v[j * NS + cs] = base + cs
    tail_base = jnp.full((NS,), NS * 2048, jnp.int32)
    for cs in range(NS):
        didx_v[NCHUNK * NS + cs] = tail_base + cs

    n_img = (IMG_SUB + IMG_HALF - 1) // IMG_HALF
    starts = [min(t * IMG_HALF, IMG_SUB - IMG_HALF) for t in range(n_img)]
    img_out = None
    for t in range(n_img):
        it = ib + starts[t]
        if img_out is not None:
            img_out.wait()
        pltpu.async_copy(img_hbm.at[pl.ds(NS * it, NS * IMG_HALF)], ibuf0,
                         smi0).wait()
        img_out = pltpu.async_copy(
            ibuf0, out_hbm.at[pl.ds(NS * (IMG_BASE + it), NS * IMG_HALF)],
            smi0)

    for j in range(NCHUNK):
        p = j % NBUF
        gat[j].wait()
        sca[j] = [
            pltpu.async_copy(
                bufs[p].at[:, pl.ds(cs * 128, 128)],
                out_hbm.at[didx_v.at[j * NS + cs]], sout[p])
            for cs in range(NS)
        ]
        k = j + NBUF
        if k < NCHUNK:
            for h in sca[j]:
                h.wait()
            gat[k] = pltpu.async_copy(table_hbm.at[sidx_v.at[k]], bufs[p],
                                      sin[p])
    for j in range(NCHUNK - NBUF, NCHUNK):
        for h in sca[j]:
            h.wait()
    img_out.wait()

    def _tail(lo, hi):
        pltpu.async_copy(table_hbm.at[sidx_v.at[NCHUNK]], buf0, si0).wait()
        tail = [
            pltpu.async_copy(buf0.at[:, pl.ds(cs * 128, 128)],
                             out_hbm.at[didx_v.at[NCHUNK * NS + cs]], so0)
            for cs in range(lo, hi)
        ]
        for h in tail:
            h.wait()

    @pl.when(w == 15)
    def _():
        _tail(0, NS // 2)

    @pl.when(w == 16)
    def _():
        _tail(NS // 2, NS)


@functools.partial(
    pl.kernel,
    mesh=plsc.VectorSubcoreMesh(core_axis_name="c", subcore_axis_name="s"),
    out_type=jax.ShapeDtypeStruct((SEQ_OUT * NS, 128), jnp.float32),
    scratch_types=[
        pltpu.VMEM((NCHUNK + 1, SUB), jnp.int32),
        pltpu.VMEM((NCHUNK * NS + NS, SUB), jnp.int32),
        pltpu.VMEM((SUB, D), jnp.float32),
        pltpu.VMEM((SUB, D), jnp.float32),
        pltpu.VMEM((SUB, D), jnp.float32),
        pltpu.VMEM((IMG_HALF * NS, 128), jnp.float32),
        pltpu.SemaphoreType.DMA,
        pltpu.SemaphoreType.DMA,
        pltpu.SemaphoreType.DMA,
        pltpu.SemaphoreType.DMA,
        pltpu.SemaphoreType.DMA,
        pltpu.SemaphoreType.DMA,
        pltpu.SemaphoreType.DMA,
    ],
)
def _sc_gather(*refs):
    _sc_body(*refs)


def kernel(embed_table, image_embeds, before_ids, after_ids, bos_id):
    bos = jnp.asarray(bos_id, jnp.int32)
    tok_src = jnp.concatenate([
        bos[None],
        before_ids[0].astype(jnp.int32),
        after_ids[0].astype(jnp.int32),
    ])
    tok3 = jnp.concatenate([
        tok_src[:2048].reshape(16, NCHUNK, SUB),
        tok_src[2049:].reshape(16, NCHUNK, SUB),
    ], axis=0)
    tok3 = jnp.concatenate([
        tok3,
        jnp.broadcast_to(tok_src[2048], (NW, 1, SUB)),
    ], axis=1)
    out = _sc_gather(
        embed_table,
        image_embeds.reshape(SEQ_IMG * NS, 128),
        tok3,
    )
    return out.reshape(1, SEQ_OUT, D)


# speedup vs baseline: 1.0457x; 1.0457x over previous
import functools

import jax
import jax.numpy as jnp
import numpy as np
from jax import lax
from jax.experimental import pallas as pl
from jax.experimental.pallas import tpu as pltpu
from jax.experimental.pallas import tpu_sc as plsc

D = 2048
NS = 16
SEQ_IMG = 729
N_TOK = 4097
SEQ_OUT = N_TOK + SEQ_IMG
IMG_BASE = 2049

NW = 32
SUB = 16
NCHUNK = 8
NBUF = 2
IMG_SUB = 23
IMG_HALF = 3

def _sc_body(table_hbm, img_hbm, tok_src_hbm, out_hbm, sidx_v,
             didx_v, buf0, buf1, ibuf0, ibuf1, si0, si1,
             so0, so1, smi0, smi1):
    c = lax.axis_index("c")
    s = lax.axis_index("s")
    w = c * 16 + s

    pltpu.sync_copy(tok_src_hbm.at[w], sidx_v)

    bufs = (buf0, buf1)
    sin = (si0, si1)
    sout = (so0, so1)
    ibufs = (ibuf0, ibuf1)
    smi = (smi0, smi1)

    ib = jnp.minimum(w * IMG_SUB, SEQ_IMG - IMG_SUB)

    gat = [None] * NCHUNK
    sca = [None] * NCHUNK
    for k in range(min(NBUF, NCHUNK)):
        gat[k] = pltpu.async_copy(
            table_hbm.at[sidx_v.at[k]], bufs[k % NBUF], sin[k % NBUF])

    dst0 = jnp.where(w < 16, w * 128, 2778 + (w - 16) * 128)
    lane = lax.iota(jnp.int32, NS) * NS
    for j in range(NCHUNK):
        base = NS * (dst0 + j * SUB) + lane
        for cs in range(NS):
            didx_v[j * NS + cs] = base + cs
    tail_base = jnp.full((NS,), NS * 2048, jnp.int32)
    for cs in range(NS):
        didx_v[NCHUNK * NS + cs] = tail_base + cs

    n_img = (IMG_SUB + IMG_HALF - 1) // IMG_HALF
    img_in = [None] * n_img
    img_out = [None] * n_img
    starts = [min(t * IMG_HALF, IMG_SUB - IMG_HALF) for t in range(n_img)]
    for t in range(min(2, n_img)):
        it = ib + starts[t]
        img_in[t] = pltpu.async_copy(
            img_hbm.at[pl.ds(NS * it, NS * IMG_HALF)], ibufs[t % 2], smi[t % 2])
    for t in range(n_img):
        it = ib + starts[t]
        img_in[t].wait()
        img_out[t] = pltpu.async_copy(
            ibufs[t % 2], out_hbm.at[pl.ds(NS * (IMG_BASE + it), NS * IMG_HALF)],
            smi[t % 2])
        if t + 2 < n_img:
            img_out[t].wait()
            img_in[t + 2] = pltpu.async_copy(
                img_hbm.at[pl.ds(NS * (ib + starts[t + 2]), NS * IMG_HALF)],
                ibufs[t % 2], smi[t % 2])

    for j in range(NCHUNK):
        p = j % NBUF
        gat[j].wait()
        sca[j] = [
            pltpu.async_copy(
                bufs[p].at[:, pl.ds(cs * 128, 128)],
                out_hbm.at[didx_v.at[j * NS + cs]], sout[p])
            for cs in range(NS)
        ]
        k = j + NBUF
        if k < NCHUNK:
            for h in sca[j]:
                h.wait()
            gat[k] = pltpu.async_copy(table_hbm.at[sidx_v.at[k]], bufs[p],
                                      sin[p])
    for j in range(NCHUNK - NBUF, NCHUNK):
        for h in sca[j]:
            h.wait()
    for t in range(max(0, n_img - 2), n_img):
        img_out[t].wait()

    def _tail(lo, hi):
        pltpu.async_copy(table_hbm.at[sidx_v.at[NCHUNK]], buf0, si0).wait()
        tail = [
            pltpu.async_copy(buf0.at[:, pl.ds(cs * 128, 128)],
                             out_hbm.at[didx_v.at[NCHUNK * NS + cs]], so0)
            for cs in range(lo, hi)
        ]
        for h in tail:
            h.wait()

    @pl.when(w == 15)
    def _():
        _tail(0, NS // 2)

    @pl.when(w == 16)
    def _():
        _tail(NS // 2, NS)


@functools.partial(
    pl.kernel,
    mesh=plsc.VectorSubcoreMesh(core_axis_name="c", subcore_axis_name="s"),
    out_type=jax.ShapeDtypeStruct((SEQ_OUT * NS, 128), jnp.float32),
    scratch_types=[
        pltpu.VMEM((NCHUNK + 1, SUB), jnp.int32),
        pltpu.VMEM((NCHUNK * NS + NS, SUB), jnp.int32),
        pltpu.VMEM((SUB, D), jnp.float32),
        pltpu.VMEM((SUB, D), jnp.float32),
        pltpu.VMEM((IMG_HALF * NS, 128), jnp.float32),
        pltpu.VMEM((IMG_HALF * NS, 128), jnp.float32),
        pltpu.SemaphoreType.DMA,
        pltpu.SemaphoreType.DMA,
        pltpu.SemaphoreType.DMA,
        pltpu.SemaphoreType.DMA,
        pltpu.SemaphoreType.DMA,
        pltpu.SemaphoreType.DMA,
    ],
)
def _sc_gather(*refs):
    _sc_body(*refs)


def kernel(embed_table, image_embeds, before_ids, after_ids, bos_id):
    bos = jnp.asarray(bos_id, jnp.int32)
    tok_src = jnp.concatenate([
        bos[None],
        before_ids[0].astype(jnp.int32),
        after_ids[0].astype(jnp.int32),
    ])
    tok3 = jnp.concatenate([
        tok_src[:2048].reshape(16, NCHUNK, SUB),
        tok_src[2049:].reshape(16, NCHUNK, SUB),
    ], axis=0)
    tok3 = jnp.concatenate([
        tok3,
        jnp.broadcast_to(tok_src[2048], (NW, 1, SUB)),
    ], axis=1)
    out = _sc_gather(
        embed_table,
        image_embeds.reshape(SEQ_IMG * NS, 128),
        tok3,
    )
    return out.reshape(1, SEQ_OUT, D)


# speedup vs baseline: 1.0772x; 1.0302x over previous
import functools

import jax
import jax.numpy as jnp
import numpy as np
from jax import lax
from jax.experimental import pallas as pl
from jax.experimental.pallas import tpu as pltpu
from jax.experimental.pallas import tpu_sc as plsc

D = 2048
NS = 16
SEQ_IMG = 729
N_TOK = 4097
SEQ_OUT = N_TOK + SEQ_IMG
IMG_BASE = 2049

NW = 32
SUB = 16
NCHUNK = 8
NBUF = 2
IMG_SUB = 23
IMG_HALF = 3

def _sc_body(table_hbm, img_hbm, tok_src_hbm, out_hbm, sidx_v,
             didx_v, buf0, buf1, ibuf0, ibuf1, si0, si1,
             so0, so1, smi0, smi1):
    c = lax.axis_index("c")
    s = lax.axis_index("s")
    w = c * 16 + s

    pltpu.sync_copy(tok_src_hbm.at[w], sidx_v)

    bufs = (buf0, buf1)
    sin = (si0, si1)
    sout = (so0, so1)
    ibufs = (ibuf0, ibuf1)
    smi = (smi0, smi1)

    ib = jnp.minimum(w * IMG_SUB, SEQ_IMG - IMG_SUB)

    gat = [None] * NCHUNK
    sca = [None] * NCHUNK
    for k in range(min(NBUF, NCHUNK)):
        gat[k] = pltpu.async_copy(
            table_hbm.at[sidx_v.at[k]], bufs[k % NBUF], sin[k % NBUF])

    dst0 = jnp.where(w < 16, w * 128, 2778 + (w - 16) * 128)
    lane = lax.iota(jnp.int32, NS) * NS
    for j in range(NCHUNK):
        base = NS * (dst0 + j * SUB) + lane
        for cs in range(NS):
            didx_v[j * NS + cs] = base + cs
    tail_base = jnp.full((NS,), NS * 2048, jnp.int32)
    for cs in range(NS):
        didx_v[NCHUNK * NS + cs] = tail_base + cs

    n_img = (IMG_SUB + IMG_HALF - 1) // IMG_HALF
    img_in = [None] * n_img
    img_out = [None] * n_img
    starts = [min(t * IMG_HALF, IMG_SUB - IMG_HALF) for t in range(n_img)]
    for t in range(min(2, n_img)):
        it = ib + starts[t]
        img_in[t] = pltpu.async_copy(
            img_hbm.at[pl.ds(NS * it, NS * IMG_HALF)], ibufs[t % 2], smi[t % 2])
    for t in range(n_img):
        it = ib + starts[t]
        img_in[t].wait()
        img_out[t] = pltpu.async_copy(
            ibufs[t % 2], out_hbm.at[pl.ds(NS * (IMG_BASE + it), NS * IMG_HALF)],
            smi[t % 2])
        if t + 2 < n_img:
            img_out[t].wait()
            img_in[t + 2] = pltpu.async_copy(
                img_hbm.at[pl.ds(NS * (ib + starts[t + 2]), NS * IMG_HALF)],
                ibufs[t % 2], smi[t % 2])

    for j in range(NCHUNK):
        p = j % NBUF
        gat[j].wait()
        sca[j] = [
            pltpu.async_copy(
                bufs[p].at[:, pl.ds(cs * 128, 128)],
                out_hbm.at[didx_v.at[j * NS + cs]], sout[p])
            for cs in range(NS)
        ]
        k = j + NBUF
        if k < NCHUNK:
            for h in sca[j]:
                h.wait()
            gat[k] = pltpu.async_copy(table_hbm.at[sidx_v.at[k]], bufs[p],
                                      sin[p])
    for j in range(NCHUNK - NBUF, NCHUNK):
        for h in sca[j]:
            h.wait()
    for t in range(max(0, n_img - 2), n_img):
        img_out[t].wait()

    def _tail(lo, hi):
        pltpu.async_copy(table_hbm.at[sidx_v.at[NCHUNK]], buf0, si0).wait()
        tail = [
            pltpu.async_copy(buf0.at[:, pl.ds(cs * 128, 128)],
                             out_hbm.at[didx_v.at[NCHUNK * NS + cs]], so0)
            for cs in range(lo, hi)
        ]
        for h in tail:
            h.wait()

    @pl.when(w == 15)
    def _():
        _tail(0, NS // 2)

    @pl.when(w == 16)
    def _():
        _tail(NS // 2, NS)


@functools.partial(
    pl.kernel,
    mesh=plsc.VectorSubcoreMesh(core_axis_name="c", subcore_axis_name="s"),
    out_type=jax.ShapeDtypeStruct((SEQ_OUT * NS, 128), jnp.float32),
    scratch_types=[
        pltpu.VMEM((NCHUNK + 1, SUB), jnp.int32),
        pltpu.VMEM((NCHUNK * NS + NS, SUB), jnp.int32),
        pltpu.VMEM((SUB, D), jnp.float32),
        pltpu.VMEM((SUB, D), jnp.float32),
        pltpu.VMEM((IMG_HALF * NS, 128), jnp.float32),
        pltpu.VMEM((IMG_HALF * NS, 128), jnp.float32),
        pltpu.SemaphoreType.DMA,
        pltpu.SemaphoreType.DMA,
        pltpu.SemaphoreType.DMA,
        pltpu.SemaphoreType.DMA,
        pltpu.SemaphoreType.DMA,
        pltpu.SemaphoreType.DMA,
    ],
)
def _sc_gather(*refs):
    _sc_body(*refs)


def kernel(embed_table, image_embeds, before_ids, after_ids, bos_id):
    bos = jnp.asarray(bos_id, jnp.int32)
    before = before_ids[0].astype(jnp.int32)
    after = after_ids[0].astype(jnp.int32)
    main = jnp.concatenate([bos[None], before[:2047], after])
    tok3 = jnp.concatenate([
        main.reshape(NW, NCHUNK * SUB),
        jnp.broadcast_to(before[2047:2048], (NW, SUB)),
    ], axis=1).reshape(NW, NCHUNK + 1, SUB)
    out = _sc_gather(
        embed_table,
        image_embeds.reshape(SEQ_IMG * NS, 128),
        tok3,
    )
    return out.reshape(1, SEQ_OUT, D)
